# baseline (speedup 1.0000x reference)
import jax
import jax.numpy as jnp
from jax.experimental import pallas as pl
from jax.experimental.pallas import tpu as pltpu

B, S, N = 4, 8, 128
D_NODE = 64
NE = N * (N - 1) // 2
C = 32
RH = 256
E = 4096
BS = B * S
NEG = -1e30


def _gat_kernel(xn_ref, xe_ref, src_ref, dst_ref, wg_ref, asrc_ref,
                adst_ref, bg_ref, g_ref):
    f32 = jnp.float32

    wg_node = wg_ref[0:D_NODE, :]
    wg_edge = wg_ref[D_NODE:, :]
    proj_n = jnp.dot(xn_ref[...], wg_node, preferred_element_type=f32)
    proj_e = jnp.dot(xe_ref[...], wg_edge, preferred_element_type=f32)

    proj = proj_n.reshape(BS, N, C) + proj_e[:, None, :]

    att_s = asrc_ref[0, :]
    att_d = adst_ref[0, :]
    a_s = jnp.sum(proj * att_s[None, None, :], axis=-1)
    a_d = jnp.sum(proj * att_d[None, None, :], axis=-1)

    iota_n = jax.lax.broadcasted_iota(jnp.int32, (E, N), 1)
    os_hot = (src_ref[0, :][:, None] == iota_n).astype(f32)
    iota_nt = jax.lax.broadcasted_iota(jnp.int32, (N, E), 0)
    od_hot_t = (dst_ref[0, :][None, :] == iota_nt).astype(f32)
    cnt = jnp.dot(od_hot_t, os_hot, preferred_element_type=f32)
    eye = (jax.lax.broadcasted_iota(jnp.int32, (N, N), 0)
           == jax.lax.broadcasted_iota(jnp.int32, (N, N), 1)).astype(f32)
    cnt = cnt + eye
    valid = cnt > 0.0

    smat = a_d[:, :, None] + a_s[:, None, :]
    smat = jnp.where(smat >= 0.0, smat, 0.2 * smat)
    m = jnp.max(jnp.where(valid[None, :, :], smat, NEG), axis=2)
    w = cnt[None, :, :] * jnp.exp(jnp.minimum(smat - m[:, :, None], 0.0))
    p = w / jnp.sum(w, axis=2)[:, :, None]

    bg = bg_ref[0, :]
    for bs in range(BS):
        o = jnp.dot(p[bs], proj[bs], preferred_element_type=f32)
        g_ref[bs * N:(bs + 1) * N, :] = jnp.maximum(o + bg[None, :], 0.0)


NIH = 6
NFC = 8
FCB = 1024


def _gru_fc_kernel(g_ref, wih_ref, whh_ref, bih_ref, bhh_ref, wfc_ref,
                   bfc_ref, out_ref, gi_ref, h_ref):
    f32 = jnp.float32
    s = pl.program_id(0)

    @pl.when(s < NIH)
    def _():
        c = jnp.minimum(s, NIH - 1)
        blk = jax.lax.dot_general(
            g_ref[...], wih_ref[...], (((1,), (1,)), ((), ())),
            preferred_element_type=f32)
        gi_ref[:, pl.ds(c * 128, 128)] = blk

    @pl.when(s == NIH)
    def _():
        gi_all = (gi_ref[...] + bih_ref[0, :][None, :]).reshape(B, S, 3 * RH)
        bhh = bhh_ref[0, :][None, :]
        h = jnp.zeros((B, RH), dtype=f32)
        for t in range(S):
            gi = gi_all[:, t, :]
            gh = jax.lax.dot_general(
                h, whh_ref[...], (((1,), (1,)), ((), ())),
                preferred_element_type=f32) + bhh
            r = jax.nn.sigmoid(gi[:, 0:RH] + gh[:, 0:RH])
            z = jax.nn.sigmoid(gi[:, RH:2 * RH] + gh[:, RH:2 * RH])
            n = jnp.tanh(gi[:, 2 * RH:] + r * gh[:, 2 * RH:])
            h = (1.0 - z) * n + z * h
        h_ref[...] = h

    @pl.when(s > NIH)
    def _():
        out_ref[...] = jnp.dot(h_ref[...], wfc_ref[...],
                               preferred_element_type=f32) \
            + bfc_ref[0, :][None, :]


@jax.jit
def kernel(x_node, x_edge, edge_index, W_gat, att_src, att_dst, b_gat,
           W_ih, W_hh, b_ih, b_hh, W_fc, b_fc):
    xn = x_node.reshape(BS * N, D_NODE)
    xe = x_edge.reshape(BS, NE)
    src = edge_index[0].astype(jnp.int32).reshape(1, E)
    dst = edge_index[1].astype(jnp.int32).reshape(1, E)

    g3 = pl.pallas_call(
        _gat_kernel,
        out_shape=jax.ShapeDtypeStruct((BS * N, C), jnp.float32),
        compiler_params=pltpu.CompilerParams(
            vmem_limit_bytes=100 * 1024 * 1024),
    )(xn, xe, src, dst, W_gat,
      att_src.reshape(1, C), att_dst.reshape(1, C), b_gat.reshape(1, C))

    g = g3.reshape(BS, N * C)

    nsteps = NIH + 1 + NFC
    out = pl.pallas_call(
        _gru_fc_kernel,
        grid=(nsteps,),
        in_specs=[
            pl.BlockSpec((BS, N * C), lambda s: (0, 0)),
            pl.BlockSpec((128, N * C), lambda s: (jnp.minimum(s, NIH - 1), 0)),
            pl.BlockSpec((3 * RH, RH), lambda s: (0, 0)),
            pl.BlockSpec((1, 3 * RH), lambda s: (0, 0)),
            pl.BlockSpec((1, 3 * RH), lambda s: (0, 0)),
            pl.BlockSpec((RH, FCB),
                         lambda s: (0, jnp.clip(s - NIH - 1, 0, NFC - 1))),
            pl.BlockSpec((1, FCB),
                         lambda s: (0, jnp.clip(s - NIH - 1, 0, NFC - 1))),
        ],
        out_specs=pl.BlockSpec(
            (B, FCB), lambda s: (0, jnp.clip(s - NIH - 1, 0, NFC - 1))),
        scratch_shapes=[
            pltpu.VMEM((BS, 3 * RH), jnp.float32),
            pltpu.VMEM((B, RH), jnp.float32),
        ],
        out_shape=jax.ShapeDtypeStruct((B, NE), jnp.float32),
        compiler_params=pltpu.CompilerParams(
            vmem_limit_bytes=100 * 1024 * 1024),
    )(g, W_ih, W_hh, b_ih.reshape(1, 3 * RH), b_hh.reshape(1, 3 * RH),
      W_fc, b_fc.reshape(1, NE))
    return out


# speedup vs baseline: 1.1243x; 1.1243x over previous
import jax
import jax.numpy as jnp
from jax.experimental import pallas as pl
from jax.experimental.pallas import tpu as pltpu

B, S, N = 4, 8, 128
D_NODE = 64
NE = N * (N - 1) // 2
C = 32
RH = 256
E = 4096
BS = B * S
NEG = -1e30

_HBM = pl.BlockSpec(memory_space=pltpu.MemorySpace.HBM)
_VMEM = pl.BlockSpec(memory_space=pltpu.MemorySpace.VMEM)


def _gat_kernel(xn_hbm, xe_hbm, src_ref, dst_ref, wg_hbm, asrc_ref,
                adst_ref, bg_ref, g_ref, xn_v, xe_v, wg_v, sems):
    f32 = jnp.float32

    cp_xn = pltpu.make_async_copy(xn_hbm, xn_v, sems.at[0])
    cp_xe = pltpu.make_async_copy(xe_hbm, xe_v, sems.at[1])
    cp_xn.start()
    cp_xe.start()
    cp_wg = []
    for k in range(4):
        cp = pltpu.make_async_copy(wg_hbm.at[pl.ds(k * 2048, 2048), :],
                                   wg_v.at[pl.ds(k * 2048, 2048), :],
                                   sems.at[2 + k])
        cp.start()
        cp_wg.append(cp)

    iota_n = jax.lax.broadcasted_iota(jnp.int32, (E, N), 1)
    os_hot = (src_ref[0, :][:, None] == iota_n).astype(f32)
    iota_nt = jax.lax.broadcasted_iota(jnp.int32, (N, E), 0)
    od_hot_t = (dst_ref[0, :][None, :] == iota_nt).astype(f32)
    cnt = jnp.dot(od_hot_t, os_hot, preferred_element_type=f32)
    eye = (jax.lax.broadcasted_iota(jnp.int32, (N, N), 0)
           == jax.lax.broadcasted_iota(jnp.int32, (N, N), 1)).astype(f32)
    cnt = cnt + eye
    valid = cnt > 0.0

    for cp in cp_wg:
        cp.wait()
    cp_xn.wait()
    cp_xe.wait()

    wg_node = wg_v[0:D_NODE, :]
    wg_edge = wg_v[D_NODE:, :]
    proj_n = jnp.dot(xn_v[...], wg_node, preferred_element_type=f32)
    proj_e = jnp.dot(xe_v[...], wg_edge, preferred_element_type=f32)

    proj = proj_n.reshape(BS, N, C) + proj_e[:, None, :]

    att_s = asrc_ref[0, :]
    att_d = adst_ref[0, :]
    a_s = jnp.sum(proj * att_s[None, None, :], axis=-1)
    a_d = jnp.sum(proj * att_d[None, None, :], axis=-1)

    smat = a_d[:, :, None] + a_s[:, None, :]
    smat = jnp.where(smat >= 0.0, smat, 0.2 * smat)
    m = jnp.max(jnp.where(valid[None, :, :], smat, NEG), axis=2)
    w = cnt[None, :, :] * jnp.exp(jnp.minimum(smat - m[:, :, None], 0.0))
    p = w / jnp.sum(w, axis=2)[:, :, None]

    bg = bg_ref[0, :]
    for bs in range(BS):
        o = jnp.dot(p[bs], proj[bs], preferred_element_type=f32)
        g_ref[bs * N:(bs + 1) * N, :] = jnp.maximum(o + bg[None, :], 0.0)


NIH = 6
NFC = 8


def _gru_fc_kernel(g_ref, wih_hbm, whh_ref, bih_ref, bhh_ref, wfc_hbm,
                   bfc_ref, out_ref, wih_v, wfc_v, sems):
    f32 = jnp.float32

    cp_ih = []
    for k in range(NIH):
        cp = pltpu.make_async_copy(wih_hbm.at[pl.ds(k * 128, 128), :],
                                   wih_v.at[pl.ds(k * 128, 128), :],
                                   sems.at[k])
        cp.start()
        cp_ih.append(cp)
    cp_fc = []
    for k in range(NFC):
        w = 1024 if k < NFC - 1 else NE - 1024 * (NFC - 1)
        cp = pltpu.make_async_copy(wfc_hbm.at[:, pl.ds(k * 1024, w)],
                                   wfc_v.at[:, pl.ds(k * 1024, w)],
                                   sems.at[NIH + k])
        cp.start()
        cp_fc.append(cp)

    gi_parts = []
    for k in range(NIH):
        cp_ih[k].wait()
        gi_parts.append(jax.lax.dot_general(
            g_ref[...], wih_v[pl.ds(k * 128, 128), :],
            (((1,), (1,)), ((), ())), preferred_element_type=f32))
    gi_all = jnp.concatenate(gi_parts, axis=1) + bih_ref[0, :][None, :]
    gi_all = gi_all.reshape(B, S, 3 * RH)
    bhh = bhh_ref[0, :][None, :]

    h = jnp.zeros((B, RH), dtype=f32)
    for t in range(S):
        gi = gi_all[:, t, :]
        gh = jax.lax.dot_general(
            h, whh_ref[...], (((1,), (1,)), ((), ())),
            preferred_element_type=f32) + bhh
        r = jax.nn.sigmoid(gi[:, 0:RH] + gh[:, 0:RH])
        z = jax.nn.sigmoid(gi[:, RH:2 * RH] + gh[:, RH:2 * RH])
        n = jnp.tanh(gi[:, 2 * RH:] + r * gh[:, 2 * RH:])
        h = (1.0 - z) * n + z * h

    for k in range(NFC):
        w = 1024 if k < NFC - 1 else NE - 1024 * (NFC - 1)
        cp_fc[k].wait()
        out_ref[:, pl.ds(k * 1024, w)] = jnp.dot(
            h, wfc_v[:, pl.ds(k * 1024, w)],
            preferred_element_type=f32) + bfc_ref[0, pl.ds(k * 1024, w)][None, :]


@jax.jit
def kernel(x_node, x_edge, edge_index, W_gat, att_src, att_dst, b_gat,
           W_ih, W_hh, b_ih, b_hh, W_fc, b_fc):
    xn = x_node.reshape(BS * N, D_NODE)
    xe = x_edge.reshape(BS, NE)
    src = edge_index[0].astype(jnp.int32).reshape(1, E)
    dst = edge_index[1].astype(jnp.int32).reshape(1, E)

    g3 = pl.pallas_call(
        _gat_kernel,
        in_specs=[_HBM, _HBM, _VMEM, _VMEM, _HBM, _VMEM, _VMEM, _VMEM],
        out_shape=jax.ShapeDtypeStruct((BS * N, C), jnp.float32),
        scratch_shapes=[
            pltpu.VMEM((BS * N, D_NODE), jnp.float32),
            pltpu.VMEM((BS, NE), jnp.float32),
            pltpu.VMEM((D_NODE + NE, C), jnp.float32),
            pltpu.SemaphoreType.DMA((6,)),
        ],
        compiler_params=pltpu.CompilerParams(
            vmem_limit_bytes=100 * 1024 * 1024),
    )(xn, xe, src, dst, W_gat,
      att_src.reshape(1, C), att_dst.reshape(1, C), b_gat.reshape(1, C))

    g = g3.reshape(BS, N * C)

    out = pl.pallas_call(
        _gru_fc_kernel,
        in_specs=[_VMEM, _HBM, _VMEM, _VMEM, _VMEM, _HBM, _VMEM],
        out_shape=jax.ShapeDtypeStruct((B, NE), jnp.float32),
        scratch_shapes=[
            pltpu.VMEM((3 * RH, N * C), jnp.float32),
            pltpu.VMEM((RH, NE), jnp.float32),
            pltpu.SemaphoreType.DMA((NIH + NFC,)),
        ],
        compiler_params=pltpu.CompilerParams(
            vmem_limit_bytes=100 * 1024 * 1024),
    )(g, W_ih, W_hh, b_ih.reshape(1, 3 * RH), b_hh.reshape(1, 3 * RH),
      W_fc, b_fc.reshape(1, NE))
    return out
